# baseline (speedup 1.0000x reference)
import jax
import jax.numpy as jnp
from jax.experimental import pallas as pl
from jax.experimental.pallas import tpu as pltpu

_T = 1024
_S = 32
_F = 512
_COLS = 1 + _S * _T
_RB = 128


def _body(x_ref, w_ref, b_ref, out_ref, scores_ref, sel_ref):
    j = pl.program_id(0)

    @pl.when(j == 0)
    def _init():
        scores = jnp.dot(x_ref[...], w_ref[...],
                         preferred_element_type=jnp.float32) + b_ref[0, 0]
        scores_ref[...] = scores
        mx = jnp.max(scores)
        rows = jax.lax.broadcasted_iota(jnp.int32, scores.shape, 0)
        sel_ref[0] = jnp.min(jnp.where(scores == mx, rows, jnp.int32(_T)))

    idx = sel_ref[0]
    cols = jax.lax.broadcasted_iota(jnp.int32, (1, _COLS), 1)
    hit = (cols > 0) & (jax.lax.rem(cols - 1, jnp.int32(_T)) == idx)
    m = jnp.where(hit, jnp.float32(-jnp.inf), jnp.float32(1.0))
    out_ref[...] = scores_ref[pl.ds(j * _RB, _RB), :] * m


def kernel(x, W, b):
    b2 = b.reshape(1, 1)
    return pl.pallas_call(
        _body,
        grid=(_T // _RB,),
        in_specs=[
            pl.BlockSpec((_T, _F), lambda j: (0, 0)),
            pl.BlockSpec((_F, 1), lambda j: (0, 0)),
            pl.BlockSpec((1, 1), lambda j: (0, 0)),
        ],
        out_specs=pl.BlockSpec((_RB, _COLS), lambda j: (j, 0)),
        out_shape=jax.ShapeDtypeStruct((_T, _COLS), jnp.float32),
        scratch_shapes=[
            pltpu.VMEM((_T, 1), jnp.float32),
            pltpu.SMEM((1,), jnp.int32),
        ],
    )(x, W, b2)
